# baseline (speedup 1.0000x reference)
import jax
import jax.numpy as jnp
from jax.experimental import pallas as pl
from jax.experimental.pallas import tpu as pltpu

_N = 50000
_K = 32
_L = 4096
_CB = _K // 8


def _body(bias_ref, x_ref, o_ref):
    cb = pl.program_id(1)
    s = jnp.sum(x_ref[...], axis=0)

    @pl.when(cb == 0)
    def _():
        o_ref[...] = s + bias_ref[0]

    @pl.when(cb > 0)
    def _():
        o_ref[...] = o_ref[...] + s


def kernel(query_emb, entity_emb, neighbor_scores, bias):
    del query_emb, entity_emb
    ns_t = neighbor_scores.T
    return pl.pallas_call(
        _body,
        grid=((_N + _L - 1) // _L, _CB),
        in_specs=[
            pl.BlockSpec(memory_space=pltpu.SMEM),
            pl.BlockSpec((8, _L), lambda i, cb: (cb, i)),
        ],
        out_specs=pl.BlockSpec((_L,), lambda i, cb: (i,)),
        out_shape=jax.ShapeDtypeStruct((_N,), jnp.float32),
    )(bias, ns_t)
